# baseline (speedup 1.0000x reference)
import jax
import jax.numpy as jnp
from jax.experimental import pallas as pl

_OUTSIZE = 512


def _zero_fill(o_ref):
    o_ref[...] = jnp.zeros_like(o_ref)


def kernel(x):
    assert x.ndim == 2
    n = x.shape[0]
    d = min(x.shape[1], _OUTSIZE)
    return pl.pallas_call(
        _zero_fill,
        out_shape=jax.ShapeDtypeStruct((n, d), jnp.float32),
    )()


# speedup vs baseline: 1.1127x; 1.1127x over previous
import jax
import jax.numpy as jnp
from jax.experimental import pallas as pl

_OUTSIZE = 512


def _zero_fill(o_ref):
    o_ref[...] = jnp.zeros_like(o_ref)


def kernel(x):
    assert x.ndim == 2
    n = x.shape[0]
    d = min(x.shape[1], _OUTSIZE)
    block_rows = min(n, 4096)
    return pl.pallas_call(
        _zero_fill,
        grid=(n // block_rows,),
        out_specs=pl.BlockSpec((block_rows, d), lambda i: (i, 0)),
        out_shape=jax.ShapeDtypeStruct((n, d), jnp.float32),
    )()


# speedup vs baseline: 1.1999x; 1.0784x over previous
import jax
import jax.numpy as jnp
from jax.experimental import pallas as pl

_OUTSIZE = 512


def _zero_fill(o_ref):
    o_ref[...] = jnp.zeros_like(o_ref)


def kernel(x):
    assert x.ndim == 2
    n = x.shape[0]
    d = min(x.shape[1], _OUTSIZE)
    block_rows = min(n, 2048)
    return pl.pallas_call(
        _zero_fill,
        grid=(n // block_rows,),
        out_specs=pl.BlockSpec((block_rows, d), lambda i: (i, 0)),
        out_shape=jax.ShapeDtypeStruct((n, d), jnp.float32),
    )()
